# baseline (speedup 1.0000x reference)
import jax
import jax.numpy as jnp
from jax.experimental import pallas as pl

_N, _D, _H, _C = 10000, 128, 128, 2
_TILE = 2000


def _mask_kernel(x_ref, w1_ref, b1_ref, a_ref, w2_ref, b2_ref, wc_ref,
                 bc_ref, gu_ref, o_ref):
    h = jnp.dot(x_ref[...], w1_ref[...], preferred_element_type=jnp.float32)
    h = h + b1_ref[...]
    a = a_ref[0, 0]
    h = jnp.where(h >= 0.0, h, a * h)
    h = jnp.dot(h, w2_ref[...], preferred_element_type=jnp.float32)
    h = h + b2_ref[...]
    s = jnp.dot(h, wc_ref[...], preferred_element_type=jnp.float32)
    s = s + bc_ref[...]
    g = -jnp.log(-jnp.log(gu_ref[...]))
    s = s + g
    m = jnp.max(s, axis=-1, keepdims=True)
    e = jnp.exp(s - m)
    y = e / jnp.sum(e, axis=-1, keepdims=True)
    hard1 = (y[:, 1:2] > y[:, 0:1]).astype(jnp.float32)
    o_ref[...] = jnp.concatenate([1.0 - hard1, hard1], axis=-1)


def kernel(x, edge_index, W1, b1, prelu_a, W2, b2, Wc, bc, gumbel_u):
    del edge_index
    b1r = b1.reshape(1, _H)
    b2r = b2.reshape(1, _H)
    bcr = bc.reshape(1, _C)
    ar = prelu_a.reshape(1, 1)
    grid = (_N // _TILE,)
    fixed = lambda i: (0, 0)
    return pl.pallas_call(
        _mask_kernel,
        grid=grid,
        in_specs=[
            pl.BlockSpec((_TILE, _D), lambda i: (i, 0)),
            pl.BlockSpec((_D, _H), fixed),
            pl.BlockSpec((1, _H), fixed),
            pl.BlockSpec((1, 1), fixed),
            pl.BlockSpec((_H, _H), fixed),
            pl.BlockSpec((1, _H), fixed),
            pl.BlockSpec((_H, _C), fixed),
            pl.BlockSpec((1, _C), fixed),
            pl.BlockSpec((_TILE, _C), lambda i: (i, 0)),
        ],
        out_specs=pl.BlockSpec((_TILE, _C), lambda i: (i, 0)),
        out_shape=jax.ShapeDtypeStruct((_N, _C), jnp.float32),
    )(x, W1, b1r, ar, W2, b2r, Wc, bcr, gumbel_u)
